# Initial kernel scaffold; baseline (speedup 1.0000x reference)
import jax
import jax.numpy as jnp
from jax.experimental import pallas as pl


def kernel(src, seg, table):
    raise NotImplementedError("write your pallas kernel here")


# speedup vs baseline: 3.6163x; 3.6163x over previous
import functools

import jax
import jax.numpy as jnp
from jax import lax
from jax.experimental import pallas as pl
from jax.experimental.pallas import tpu as pltpu
from jax.experimental.pallas import tpu_sc as plsc

BATCH = 4
SEQ = 8192
EMB = 1024
NUM_CORES = 2
NUM_SUBCORES = 16
NUM_WORKERS = NUM_CORES * NUM_SUBCORES
ROWS_PER_WORKER = SEQ // NUM_WORKERS
CHUNK_ROWS = 64
NUM_CHUNKS = ROWS_PER_WORKER // CHUNK_ROWS

_mesh = plsc.VectorSubcoreMesh(core_axis_name="c", subcore_axis_name="s")


@functools.partial(
    pl.kernel,
    mesh=_mesh,
    out_type=jax.ShapeDtypeStruct((BATCH, SEQ, EMB), jnp.float32),
    scratch_types=[pltpu.VMEM((CHUNK_ROWS, EMB), jnp.float32)],
)
def _broadcast_table(table_hbm, out_hbm, buf):
    wid = lax.axis_index("s") * NUM_CORES + lax.axis_index("c")
    base = wid * ROWS_PER_WORKER
    for i in range(NUM_CHUNKS):
        row = base + i * CHUNK_ROWS
        pltpu.sync_copy(table_hbm.at[pl.ds(row, CHUNK_ROWS)], buf)
        for b in range(BATCH):
            pltpu.sync_copy(buf, out_hbm.at[b, pl.ds(row, CHUNK_ROWS)])


def kernel(src, seg, table):
    del src, seg
    return _broadcast_table(table)
